# Initial kernel scaffold; baseline (speedup 1.0000x reference)
import jax
import jax.numpy as jnp
from jax.experimental import pallas as pl


def kernel(input_ids, embeddings):
    raise NotImplementedError("write your pallas kernel here")


# speedup vs baseline: 1.1714x; 1.1714x over previous
import functools

import jax
import jax.numpy as jnp
from jax import lax
from jax.experimental import pallas as pl
from jax.experimental.pallas import tpu as pltpu
from jax.experimental.pallas import tpu_sc as plsc

VOCAB = 28996
DIM = 768


_BLK = 1024


def _stats_body(emb_ref, sum_ref, sq_ref):
    i = pl.program_id(0)
    x = emb_ref[...]
    row = lax.broadcasted_iota(jnp.int32, x.shape, 0) + i * _BLK
    x = jnp.where(row < VOCAB, x, 0.0)
    s = jnp.sum(x, axis=0, keepdims=True)
    q = jnp.sum(x * x, axis=0, keepdims=True)

    @pl.when(i == 0)
    def _():
        sum_ref[...] = s
        sq_ref[...] = q

    @pl.when(i > 0)
    def _():
        sum_ref[...] += s
        sq_ref[...] += q


def _column_stats(embeddings):
    grid = (VOCAB + _BLK - 1) // _BLK
    s, q = pl.pallas_call(
        _stats_body,
        grid=(grid,),
        in_specs=[pl.BlockSpec((_BLK, DIM), lambda i: (i, 0))],
        out_specs=[
            pl.BlockSpec((1, DIM), lambda i: (0, 0)),
            pl.BlockSpec((1, DIM), lambda i: (0, 0)),
        ],
        out_shape=[
            jax.ShapeDtypeStruct((1, DIM), jnp.float32),
            jax.ShapeDtypeStruct((1, DIM), jnp.float32),
        ],
    )(embeddings)
    n = jnp.float32(VOCAB)
    mean = s[0] / n
    var = (q[0] - s[0] * s[0] / n) / (n - 1.0)
    rstd = lax.rsqrt(var)
    return mean, rstd


_NW = 32
_L = 16
_CH = 64


def _make_gather_norm(B):
    bpw = B // _NW
    nch = bpw // _CH
    mesh = plsc.VectorSubcoreMesh(core_axis_name="c", subcore_axis_name="s")

    @functools.partial(
        pl.kernel,
        mesh=mesh,
        out_type=jax.ShapeDtypeStruct((B, DIM), jnp.float32),
        scratch_types=[
            pltpu.VMEM((bpw,), jnp.int32),
            pltpu.VMEM((_CH, DIM), jnp.float32),
            pltpu.VMEM((DIM,), jnp.float32),
            pltpu.VMEM((DIM,), jnp.float32),
            pltpu.SemaphoreType.DMA,
        ],
    )
    def gather_norm(table_hbm, ids_hbm, mean_hbm, rstd_hbm, out_hbm,
                    idx_v, rows_v, mean_v, rstd_v, sem):
        wid = lax.axis_index("s") * 2 + lax.axis_index("c")
        base = wid * bpw
        pltpu.sync_copy(ids_hbm.at[pl.ds(base, bpw)], idx_v)
        pltpu.sync_copy(mean_hbm, mean_v)
        pltpu.sync_copy(rstd_hbm, rstd_v)

        def chunk(c, carry):
            pltpu.async_copy(
                table_hbm.at[idx_v.at[pl.ds(c * _CH, _CH)]], rows_v, sem
            ).wait()
            for j in range(DIM // _L):
                mj = mean_v[pl.ds(j * _L, _L)]
                rj = rstd_v[pl.ds(j * _L, _L)]

                def row(r, carry2):
                    x = rows_v[r, pl.ds(j * _L, _L)]
                    rows_v[r, pl.ds(j * _L, _L)] = (x - mj) * rj
                    return carry2

                lax.fori_loop(0, _CH, row, 0, unroll=8)
            pltpu.sync_copy(rows_v, out_hbm.at[pl.ds(base + c * _CH, _CH)])
            return carry

        lax.fori_loop(0, nch, chunk, 0)

    return gather_norm


def kernel(input_ids, embeddings):
    ids_flat = input_ids.reshape(-1).astype(jnp.int32)
    B = ids_flat.shape[0]
    mean, rstd = _column_stats(embeddings)
    out = _make_gather_norm(B)(embeddings, ids_flat, mean, rstd)
    return out.reshape(*input_ids.shape, DIM)
